# baseline (speedup 1.0000x reference)
import functools

import jax
import jax.numpy as jnp
from jax import lax
from jax.experimental import pallas as pl
from jax.experimental.pallas import tpu as pltpu
from jax.experimental.pallas import tpu_sc as plsc

_DIM = 128
_NUM_POS = 100000

_ROW_BLOCK = 4000
_PI = 3.141592653589793
_HALF_PI = _PI / 2


def _fast_sin(x):
    sign = jnp.where(x > _PI, -1.0, 1.0)
    y = jnp.where(x > _PI, x - _PI, x)
    y = jnp.where(y > _HALF_PI, _PI - y, y)
    s = y * y
    p = jnp.float32(1.0 / 362880)
    p = p * s + jnp.float32(-1.0 / 5040)
    p = p * s + jnp.float32(1.0 / 120)
    p = p * s + jnp.float32(-1.0 / 6)
    p = p * s + 1.0
    return sign * y * p


def _table_body(out_ref):
    i = pl.program_id(0)
    r = jax.lax.broadcasted_iota(jnp.int32, (_ROW_BLOCK, _DIM), 0)
    r = (r + i * _ROW_BLOCK).astype(jnp.float32)
    b = r * jnp.float32(1.0 / 10000.0)
    c = jax.lax.broadcasted_iota(jnp.int32, (_ROW_BLOCK, _DIM), 1)
    k = c // 2
    e = k.astype(jnp.float32) * jnp.float32(1.0 / _DIM)
    phase = jnp.where(k == 0, 1.0, jnp.exp2(e * jnp.log2(b)))
    phase = phase + jnp.where(c % 2 == 0, 0.0, _HALF_PI)
    out_ref[...] = _fast_sin(phase)


def _build_table():
    return pl.pallas_call(
        _table_body,
        out_shape=jax.ShapeDtypeStruct((_NUM_POS, _DIM), jnp.float32),
        grid=(_NUM_POS // _ROW_BLOCK,),
        out_specs=pl.BlockSpec((_ROW_BLOCK, _DIM), lambda i: (i, 0)),
    )()


_T_ROWS = 16384
_T_COLS = 50
_B = _T_ROWS * _T_COLS
_NW = 32
_I_PER_W = _T_ROWS // _NW
_H = _I_PER_W // 2


_NBUF = 4


def _gather_kernel(
    table_hbm, idx_hbm, out_hbm, idx_a, idx_b,
    rows_0, rows_1, rows_2, rows_3, isem_a, isem_b,
    gsem_0, gsem_1, gsem_2, gsem_3, osem_0, osem_1, osem_2, osem_3
):
    nc = 2
    wid = lax.axis_index("s") * nc + lax.axis_index("c")
    base_i = wid * _I_PER_W
    rows = (rows_0, rows_1, rows_2, rows_3)
    gsems = (gsem_0, gsem_1, gsem_2, gsem_3)
    osems = (osem_0, osem_1, osem_2, osem_3)

    def do_column(j, idx_v):
        handles = []
        for u in range(_NBUF):
            @pl.when(j > 0)
            def _(u=u):
                pltpu.make_async_copy(
                    rows[u],
                    out_hbm.at[
                        pl.ds((j - 1) * _T_ROWS + base_i + u * _DIM, _DIM)
                    ],
                    osems[u],
                ).wait()

            handles.append(
                pltpu.async_copy(
                    table_hbm.at[idx_v.at[pl.ds(u * _DIM, _DIM)]],
                    rows[u],
                    gsems[u],
                )
            )
        for u in range(_NBUF):
            handles[u].wait()
            pltpu.async_copy(
                rows[u],
                out_hbm.at[pl.ds(j * _T_ROWS + base_i + u * _DIM, _DIM)],
                osems[u],
            )

    pltpu.async_copy(idx_hbm.at[pl.ds(base_i, _I_PER_W)], idx_a, isem_a)
    pltpu.async_copy(
        idx_hbm.at[pl.ds(_T_ROWS + base_i, _I_PER_W)], idx_b, isem_b
    )

    def column_pair(p, carry):
        for j, idx_v, isem in ((2 * p, idx_a, isem_a), (2 * p + 1, idx_b, isem_b)):
            pltpu.make_async_copy(
                idx_hbm.at[pl.ds(j * _T_ROWS + base_i, _I_PER_W)], idx_v, isem
            ).wait()
            do_column(j, idx_v)

            @pl.when(p < _T_COLS // 2 - 1)
            def _(j=j, idx_v=idx_v, isem=isem):
                pltpu.async_copy(
                    idx_hbm.at[pl.ds((j + 2) * _T_ROWS + base_i, _I_PER_W)],
                    idx_v,
                    isem,
                )

        return carry

    lax.fori_loop(0, _T_COLS // 2, column_pair, 0)
    j_last = _T_COLS - 1
    for u in range(_NBUF):
        pltpu.make_async_copy(
            rows[u],
            out_hbm.at[pl.ds(j_last * _T_ROWS + base_i + u * _DIM, _DIM)],
            osems[u],
        ).wait()


def _gather(table, idx_flat):
    mesh = plsc.VectorSubcoreMesh(core_axis_name="c", subcore_axis_name="s")
    f = functools.partial(
        pl.kernel,
        mesh=mesh,
        out_type=jax.ShapeDtypeStruct((_B, _DIM), jnp.float32),
        scratch_types=(
            [pltpu.VMEM((_I_PER_W,), jnp.int32)] * 2
            + [pltpu.VMEM((_DIM, _DIM), jnp.float32)] * _NBUF
            + [pltpu.SemaphoreType.DMA] * (2 + 2 * _NBUF)
        ),
    )(_gather_kernel)
    return f(table, idx_flat)


def kernel(t):
    idx = t.T.astype(jnp.int32).reshape(-1)
    table = _build_table()
    out = _gather(table, idx)
    return out.reshape(_T_COLS, _T_ROWS, _DIM).transpose(1, 0, 2)
